# baseline (speedup 1.0000x reference)
import functools

import jax
import jax.numpy as jnp
from jax import lax
from jax.experimental import pallas as pl
from jax.experimental.pallas import tpu as pltpu
from jax.experimental.pallas import tpu_sc as plsc

EPS = 1e-5
LANES = 16
K = 128
CH = 128

_SC_PARAMS = pltpu.CompilerParams(
    needs_layout_passes=False, use_tc_tiling_on_sc=False)


def _sc_geometry():
    try:
        info = plsc.get_sparse_core_info()
        return int(info.num_cores), int(info.num_subcores)
    except Exception:
        return 2, 16


def _make_degree_kernel(NC, NSUB, NP, EPW):
    NW = NC * NSUB
    mesh = plsc.VectorSubcoreMesh(core_axis_name="c", subcore_axis_name="s")

    @functools.partial(
        pl.kernel,
        mesh=mesh,
        out_type=jax.ShapeDtypeStruct((NW, NP), jnp.float32),
        scratch_types=[
            pltpu.VMEM((EPW,), jnp.int32),
            pltpu.VMEM((NP,), jnp.float32),
        ],
        compiler_params=_SC_PARAMS,
    )
    def deg_kernel(dst_hbm, zeros_hbm, out_hbm, dst_v, hist_v):
        c = lax.axis_index("c")
        s = lax.axis_index("s")
        w = c * NSUB + s
        pltpu.sync_copy(dst_hbm.at[w], dst_v)
        pltpu.sync_copy(zeros_hbm, hist_v)
        ones = jnp.ones((LANES,), jnp.float32)

        def body(i, carry):
            idx = dst_v[pl.ds(i * LANES, LANES)]
            plsc.addupdate_scatter(hist_v, [idx], ones)
            return carry

        lax.fori_loop(0, EPW // LANES, body, 0)
        pltpu.sync_copy(hist_v, out_hbm.at[w])

    return deg_kernel


def _pipelined_agg(h_ref, src_v, dst_v, rows0, rows1, acc_sh,
                   g0, g1, s0, s1, NB):
    def g_start(j, rows, sem):
        pltpu.async_copy(h_ref.at[src_v.at[j]], rows, sem)

    def g_wait(j, rows, sem):
        pltpu.make_async_copy(h_ref.at[src_v.at[j]], rows, sem).wait()

    def s_start(j, rows, sem):
        pltpu.async_copy(rows, acc_sh.at[dst_v.at[j]], sem, add=True)

    def s_wait(j, rows, sem):
        pltpu.make_async_copy(rows, acc_sh.at[dst_v.at[j]], sem).wait()

    NH = NB // 2
    g_start(0, rows0, g0)

    def body(i, carry):
        j0 = 2 * i
        j1 = j0 + 1
        g_wait(j0, rows0, g0)
        s_start(j0, rows0, s0)
        g_start(j1, rows1, g1)
        g_wait(j1, rows1, g1)
        s_wait(j0, rows0, s0)
        s_start(j1, rows1, s1)
        g_start(j0 + 2, rows0, g0)
        s_wait(j1, rows1, s1)
        return carry

    lax.fori_loop(0, NH - 1, body, 0)
    jl0 = NB - 2
    jl1 = NB - 1
    g_wait(jl0, rows0, g0)
    s_start(jl0, rows0, s0)
    g_start(jl1, rows1, g1)
    g_wait(jl1, rows1, g1)
    s_wait(jl0, rows0, s0)
    s_start(jl1, rows1, s1)
    s_wait(jl1, rows1, s1)


def _make_agg_colsplit_kernel(NC, NSUB, NP, D, NB):
    RPT = NP // NSUB
    NCH = RPT // CH
    mesh = plsc.VectorSubcoreMesh(core_axis_name="c", subcore_axis_name="s")

    @functools.partial(
        pl.kernel,
        mesh=mesh,
        out_type=jax.ShapeDtypeStruct((NC, NP, D), jnp.float32),
        scratch_types=[
            pltpu.VMEM((NB, K), jnp.int32),
            pltpu.VMEM((NB, K), jnp.int32),
            pltpu.VMEM((K, D), jnp.float32),
            pltpu.VMEM((K, D), jnp.float32),
            pltpu.VMEM((CH, D), jnp.float32),
            pltpu.VMEM_SHARED((NP, D), jnp.float32),
            pltpu.SemaphoreType.DMA,
            pltpu.SemaphoreType.DMA,
            pltpu.SemaphoreType.DMA,
            pltpu.SemaphoreType.DMA,
        ],
        compiler_params=_SC_PARAMS,
    )
    def agg_kernel(src_hbm, dst_hbm, hsa_hbm, hsb_hbm, zrows_hbm, out_hbm,
                   src_v, dst_v, rows0, rows1, bounce_v, acc_sh,
                   g0, g1, s0, s1):
        c = lax.axis_index("c")
        s = lax.axis_index("s")
        pltpu.sync_copy(src_hbm.at[s], src_v)
        pltpu.sync_copy(dst_hbm.at[s], dst_v)
        pltpu.sync_copy(zrows_hbm, bounce_v)
        for k in range(NCH):
            pltpu.sync_copy(bounce_v, acc_sh.at[pl.ds(s * RPT + k * CH, CH)])
        plsc.subcore_barrier()

        @pl.when(c == 0)
        def _():
            _pipelined_agg(hsa_hbm, src_v, dst_v, rows0, rows1, acc_sh,
                           g0, g1, s0, s1, NB)

        @pl.when(c != 0)
        def _():
            _pipelined_agg(hsb_hbm, src_v, dst_v, rows0, rows1, acc_sh,
                           g0, g1, s0, s1, NB)

        plsc.subcore_barrier()
        for k in range(NCH):
            pltpu.sync_copy(acc_sh.at[pl.ds(s * RPT + k * CH, CH)], bounce_v)
            pltpu.sync_copy(bounce_v, out_hbm.at[c, pl.ds(s * RPT + k * CH, CH)])

    return agg_kernel


def _make_agg_kernel(NC, NSUB, NP, D, NB):
    RPT = NP // NSUB
    NCH = RPT // CH
    mesh = plsc.VectorSubcoreMesh(core_axis_name="c", subcore_axis_name="s")

    @functools.partial(
        pl.kernel,
        mesh=mesh,
        out_type=jax.ShapeDtypeStruct((NC, NP, D), jnp.float32),
        scratch_types=[
            pltpu.VMEM((NB, K), jnp.int32),
            pltpu.VMEM((NB, K), jnp.int32),
            pltpu.VMEM((K, D), jnp.float32),
            pltpu.VMEM((K, D), jnp.float32),
            pltpu.VMEM((CH, D), jnp.float32),
            pltpu.VMEM_SHARED((NP, D), jnp.float32),
            pltpu.SemaphoreType.DMA,
            pltpu.SemaphoreType.DMA,
            pltpu.SemaphoreType.DMA,
            pltpu.SemaphoreType.DMA,
        ],
        compiler_params=_SC_PARAMS,
    )
    def agg_kernel(src_hbm, dst_hbm, hs_hbm, zrows_hbm, out_hbm,
                   src_v, dst_v, rows0, rows1, bounce_v, acc_sh,
                   g0, g1, s0, s1):
        c = lax.axis_index("c")
        s = lax.axis_index("s")
        w = c * NSUB + s
        pltpu.sync_copy(src_hbm.at[w], src_v)
        pltpu.sync_copy(dst_hbm.at[w], dst_v)
        pltpu.sync_copy(zrows_hbm, bounce_v)
        for k in range(NCH):
            pltpu.sync_copy(bounce_v, acc_sh.at[pl.ds(s * RPT + k * CH, CH)])
        plsc.subcore_barrier()

        _pipelined_agg(hs_hbm, src_v, dst_v, rows0, rows1, acc_sh,
                       g0, g1, s0, s1, NB)
        plsc.subcore_barrier()
        for k in range(NCH):
            pltpu.sync_copy(acc_sh.at[pl.ds(s * RPT + k * CH, CH)], bounce_v)
            pltpu.sync_copy(bounce_v, out_hbm.at[c, pl.ds(s * RPT + k * CH, CH)])

    return agg_kernel


def _tc_deg_matmul(x_p, W, cvec, hists, NW, BR):
    NP, Din = x_p.shape
    Dout = W.shape[1]
    DH = Dout // 2

    def body(x_ref, w_ref, c_ref, h_ref, hsa_ref, hsb_ref, dinv_ref):
        deg = jnp.sum(h_ref[...], axis=0) + 1.0
        dinv = lax.rsqrt(deg)[:, None]
        h = jnp.dot(x_ref[...], w_ref[...], preferred_element_type=jnp.float32)
        hs = h * c_ref[...] * dinv
        hsa_ref[...] = hs[:, :DH]
        hsb_ref[...] = hs[:, DH:]
        dinv_ref[...] = dinv

    return pl.pallas_call(
        body,
        grid=(NP // BR,),
        in_specs=[
            pl.BlockSpec((BR, Din), lambda i: (i, 0)),
            pl.BlockSpec((Din, Dout), lambda i: (0, 0)),
            pl.BlockSpec((1, Dout), lambda i: (0, 0)),
            pl.BlockSpec((NW, BR), lambda i: (0, i)),
        ],
        out_specs=[
            pl.BlockSpec((BR, DH), lambda i: (i, 0)),
            pl.BlockSpec((BR, DH), lambda i: (i, 0)),
            pl.BlockSpec((BR, 1), lambda i: (i, 0)),
        ],
        out_shape=[
            jax.ShapeDtypeStruct((NP, DH), jnp.float32),
            jax.ShapeDtypeStruct((NP, DH), jnp.float32),
            jax.ShapeDtypeStruct((NP, 1), jnp.float32),
        ],
    )(x_p, W, cvec, hists)


def _tc_mid(za, zb, hsa, hsb, dinv, dvec, W, cvec, N, BR):
    NP, DH = hsa.shape
    Din = 2 * DH
    Dout = W.shape[1]

    def body(za_ref, zb_ref, hsa_ref, hsb_ref, dinv_ref, d_ref, w_ref, c_ref,
             out_ref):
        i = pl.program_id(0)
        di = dinv_ref[...]
        xa = (za_ref[...] + hsa_ref[...]) * di + d_ref[:, :DH]
        xb = (zb_ref[...] + hsb_ref[...]) * di + d_ref[:, DH:]
        x2 = jnp.maximum(jnp.concatenate([xa, xb], axis=1), 0.0)
        rid = i * BR + lax.broadcasted_iota(jnp.int32, (BR, Din), 0)
        x2 = jnp.where(rid < N, x2, 0.0)
        h = jnp.dot(x2, w_ref[...], preferred_element_type=jnp.float32)
        out_ref[...] = h * c_ref[...] * di

    return pl.pallas_call(
        body,
        grid=(NP // BR,),
        in_specs=[
            pl.BlockSpec((BR, DH), lambda i: (i, 0)),
            pl.BlockSpec((BR, DH), lambda i: (i, 0)),
            pl.BlockSpec((BR, DH), lambda i: (i, 0)),
            pl.BlockSpec((BR, DH), lambda i: (i, 0)),
            pl.BlockSpec((BR, 1), lambda i: (i, 0)),
            pl.BlockSpec((1, Din), lambda i: (0, 0)),
            pl.BlockSpec((Din, Dout), lambda i: (0, 0)),
            pl.BlockSpec((1, Dout), lambda i: (0, 0)),
        ],
        out_specs=pl.BlockSpec((BR, Dout), lambda i: (i, 0)),
        out_shape=jax.ShapeDtypeStruct((NP, Dout), jnp.float32),
    )(za, zb, hsa, hsb, dinv, dvec, W, cvec)


def _tc_final(za, zb, hs2, dinv, dvec, N, BC):
    D = hs2.shape[1]

    def body(za_ref, zb_ref, hs_ref, dinv_ref, d_ref, out_ref):
        out_ref[...] = ((za_ref[...] + zb_ref[...] + hs_ref[...])
                        * dinv_ref[...] + d_ref[...])

    return pl.pallas_call(
        body,
        grid=(N // BC,),
        in_specs=[
            pl.BlockSpec((BC, D), lambda i: (i, 0)),
            pl.BlockSpec((BC, D), lambda i: (i, 0)),
            pl.BlockSpec((BC, D), lambda i: (i, 0)),
            pl.BlockSpec((BC, 1), lambda i: (i, 0)),
            pl.BlockSpec((1, D), lambda i: (0, 0)),
        ],
        out_specs=pl.BlockSpec((BC, D), lambda i: (i, 0)),
        out_shape=jax.ShapeDtypeStruct((N, D), jnp.float32),
    )(za, zb, hs2, dinv, dvec)


def kernel(x, edge_index, W1, b1, g1, beta1, rm1, rv1,
           W2, b2, g2, beta2, rm2, rv2):
    N, Din = x.shape
    D1 = W1.shape[1]
    D2 = W2.shape[1]
    E = edge_index.shape[1]
    NC, NSUB = _sc_geometry()
    NW = NC * NSUB

    unit = NSUB * CH
    NP = -(-(N + 1) // unit) * unit
    NB2 = -(-E // (NW * K))
    NB2 += NB2 % 2
    NB1 = NC * NB2
    EP = NW * NB2 * K

    src = edge_index[0].astype(jnp.int32)
    dst = edge_index[1].astype(jnp.int32)
    pad = jnp.full((EP - E,), N, jnp.int32)
    src_p = jnp.concatenate([src, pad])
    dst_p = jnp.concatenate([dst, pad])
    x_p = jnp.pad(x, ((0, NP - N), (0, 0)))

    c1 = g1 * lax.rsqrt(rv1 + EPS)
    d1 = (beta1 + (b1 - rm1) * c1)[None, :]
    c1 = c1[None, :]
    c2 = g2 * lax.rsqrt(rv2 + EPS)
    d2 = (beta2 + (b2 - rm2) * c2)[None, :]
    c2 = c2[None, :]

    zeros_np = jnp.zeros((NP,), jnp.float32)
    zrows1 = jnp.zeros((CH, D1 // NC), jnp.float32)
    zrows2 = jnp.zeros((CH, D2), jnp.float32)

    BR = NP // 10 if NP % 10 == 0 else NP // 8
    BC = N // 10 if N % 10 == 0 else N // 8

    hists = _make_degree_kernel(NC, NSUB, NP, NB2 * K)(
        dst_p.reshape(NW, NB2 * K), zeros_np)
    hsa, hsb, dinv = _tc_deg_matmul(x_p, W1, c1, hists, NW, BR)
    z1 = _make_agg_colsplit_kernel(NC, NSUB, NP, D1 // NC, NB1)(
        src_p.reshape(NSUB, NB1, K), dst_p.reshape(NSUB, NB1, K),
        hsa, hsb, zrows1)
    hs2 = _tc_mid(z1[0], z1[1], hsa, hsb, dinv, d1, W2, c2, N, BR)
    z2 = _make_agg_kernel(NC, NSUB, NP, D2, NB2)(
        src_p.reshape(NW, NB2, K), dst_p.reshape(NW, NB2, K), hs2, zrows2)
    return _tc_final(z2[0], z2[1], hs2, dinv, d2, N, BC)


# speedup vs baseline: 1.2456x; 1.2456x over previous
import functools

import jax
import jax.numpy as jnp
from jax import lax
from jax.experimental import pallas as pl
from jax.experimental.pallas import tpu as pltpu
from jax.experimental.pallas import tpu_sc as plsc

EPS = 1e-5
LANES = 16
K = 128
CH = 128

_SC_PARAMS = pltpu.CompilerParams(
    needs_layout_passes=False, use_tc_tiling_on_sc=False)


def _sc_geometry():
    try:
        info = plsc.get_sparse_core_info()
        return int(info.num_cores), int(info.num_subcores)
    except Exception:
        return 2, 16


def _make_degree_kernel(NC, NSUB, NP, EPW):
    NW = NC * NSUB
    mesh = plsc.VectorSubcoreMesh(core_axis_name="c", subcore_axis_name="s")

    @functools.partial(
        pl.kernel,
        mesh=mesh,
        out_type=jax.ShapeDtypeStruct((NW, NP), jnp.float32),
        scratch_types=[
            pltpu.VMEM((EPW,), jnp.int32),
            pltpu.VMEM((NP,), jnp.float32),
        ],
        compiler_params=_SC_PARAMS,
    )
    def deg_kernel(dst_hbm, zeros_hbm, out_hbm, dst_v, hist_v):
        c = lax.axis_index("c")
        s = lax.axis_index("s")
        w = c * NSUB + s
        pltpu.sync_copy(dst_hbm.at[w], dst_v)
        pltpu.sync_copy(zeros_hbm, hist_v)
        ones = jnp.ones((LANES,), jnp.float32)

        def body(i, carry):
            idx = dst_v[pl.ds(i * LANES, LANES)]
            plsc.addupdate_scatter(hist_v, [idx], ones)
            return carry

        lax.fori_loop(0, EPW // LANES, body, 0)
        pltpu.sync_copy(hist_v, out_hbm.at[w])

    return deg_kernel


def _stream_agg(h_ref, src_v, dst_v, rows0, acc_sh, g0, NB):
    def body(j, carry):
        pltpu.async_copy(h_ref.at[src_v.at[j]], rows0, g0).wait()
        pltpu.sync_copy(rows0, acc_sh.at[dst_v.at[j]], add=True)
        return carry

    lax.fori_loop(0, NB, body, 0)


def _agg_scratch(NB, D, NP):
    return [
        pltpu.VMEM((NB, K), jnp.int32),
        pltpu.VMEM((NB, K), jnp.int32),
        pltpu.VMEM((K, D), jnp.float32),
        pltpu.VMEM((CH, D), jnp.float32),
        pltpu.VMEM_SHARED((NP, D), jnp.float32),
        pltpu.SemaphoreType.DMA,
    ]


def _make_agg_colsplit_kernel(NC, NSUB, NP, D, NB):
    RPT = NP // NSUB
    NCH = RPT // CH
    mesh = plsc.VectorSubcoreMesh(core_axis_name="c", subcore_axis_name="s")

    @functools.partial(
        pl.kernel,
        mesh=mesh,
        out_type=jax.ShapeDtypeStruct((NC, NP, D), jnp.float32),
        scratch_types=_agg_scratch(NB, D, NP),
        compiler_params=_SC_PARAMS,
    )
    def agg_kernel(src_hbm, dst_hbm, hsa_hbm, hsb_hbm, zrows_hbm, out_hbm,
                   src_v, dst_v, rows0, bounce_v, acc_sh, g0):
        c = lax.axis_index("c")
        s = lax.axis_index("s")
        pltpu.sync_copy(src_hbm.at[s], src_v)
        pltpu.sync_copy(dst_hbm.at[s], dst_v)
        pltpu.sync_copy(zrows_hbm, bounce_v)
        for k in range(NCH):
            pltpu.sync_copy(bounce_v, acc_sh.at[pl.ds(s * RPT + k * CH, CH)])
        plsc.subcore_barrier()

        @pl.when(c == 0)
        def _():
            _stream_agg(hsa_hbm, src_v, dst_v, rows0, acc_sh, g0, NB)

        @pl.when(c != 0)
        def _():
            _stream_agg(hsb_hbm, src_v, dst_v, rows0, acc_sh, g0, NB)

        plsc.subcore_barrier()
        for k in range(NCH):
            pltpu.sync_copy(acc_sh.at[pl.ds(s * RPT + k * CH, CH)], bounce_v)
            pltpu.sync_copy(bounce_v, out_hbm.at[c, pl.ds(s * RPT + k * CH, CH)])

    return agg_kernel


def _make_agg_kernel(NC, NSUB, NP, D, NB):
    RPT = NP // NSUB
    NCH = RPT // CH
    mesh = plsc.VectorSubcoreMesh(core_axis_name="c", subcore_axis_name="s")

    @functools.partial(
        pl.kernel,
        mesh=mesh,
        out_type=jax.ShapeDtypeStruct((NC, NP, D), jnp.float32),
        scratch_types=_agg_scratch(NB, D, NP),
        compiler_params=_SC_PARAMS,
    )
    def agg_kernel(src_hbm, dst_hbm, hs_hbm, zrows_hbm, out_hbm,
                   src_v, dst_v, rows0, bounce_v, acc_sh, g0):
        c = lax.axis_index("c")
        s = lax.axis_index("s")
        w = c * NSUB + s
        pltpu.sync_copy(src_hbm.at[w], src_v)
        pltpu.sync_copy(dst_hbm.at[w], dst_v)
        pltpu.sync_copy(zrows_hbm, bounce_v)
        for k in range(NCH):
            pltpu.sync_copy(bounce_v, acc_sh.at[pl.ds(s * RPT + k * CH, CH)])
        plsc.subcore_barrier()

        _stream_agg(hs_hbm, src_v, dst_v, rows0, acc_sh, g0, NB)
        plsc.subcore_barrier()
        for k in range(NCH):
            pltpu.sync_copy(acc_sh.at[pl.ds(s * RPT + k * CH, CH)], bounce_v)
            pltpu.sync_copy(bounce_v, out_hbm.at[c, pl.ds(s * RPT + k * CH, CH)])

    return agg_kernel


def _tc_deg_matmul(x_p, W, cvec, hists, NW, BR):
    NP, Din = x_p.shape
    Dout = W.shape[1]
    DH = Dout // 2

    def body(x_ref, w_ref, c_ref, h_ref, hsa_ref, hsb_ref, dinv_ref):
        deg = jnp.sum(h_ref[...], axis=0) + 1.0
        dinv = lax.rsqrt(deg)[:, None]
        h = jnp.dot(x_ref[...], w_ref[...], preferred_element_type=jnp.float32)
        hs = h * c_ref[...] * dinv
        hsa_ref[...] = hs[:, :DH]
        hsb_ref[...] = hs[:, DH:]
        dinv_ref[...] = dinv

    return pl.pallas_call(
        body,
        grid=(NP // BR,),
        in_specs=[
            pl.BlockSpec((BR, Din), lambda i: (i, 0)),
            pl.BlockSpec((Din, Dout), lambda i: (0, 0)),
            pl.BlockSpec((1, Dout), lambda i: (0, 0)),
            pl.BlockSpec((NW, BR), lambda i: (0, i)),
        ],
        out_specs=[
            pl.BlockSpec((BR, DH), lambda i: (i, 0)),
            pl.BlockSpec((BR, DH), lambda i: (i, 0)),
            pl.BlockSpec((BR, 1), lambda i: (i, 0)),
        ],
        out_shape=[
            jax.ShapeDtypeStruct((NP, DH), jnp.float32),
            jax.ShapeDtypeStruct((NP, DH), jnp.float32),
            jax.ShapeDtypeStruct((NP, 1), jnp.float32),
        ],
    )(x_p, W, cvec, hists)


def _tc_mid(za, zb, hsa, hsb, dinv, dvec, W, cvec, N, BR):
    NP, DH = hsa.shape
    Din = 2 * DH
    Dout = W.shape[1]

    def body(za_ref, zb_ref, hsa_ref, hsb_ref, dinv_ref, d_ref, w_ref, c_ref,
             out_ref):
        i = pl.program_id(0)
        di = dinv_ref[...]
        xa = (za_ref[...] + hsa_ref[...]) * di + d_ref[:, :DH]
        xb = (zb_ref[...] + hsb_ref[...]) * di + d_ref[:, DH:]
        x2 = jnp.maximum(jnp.concatenate([xa, xb], axis=1), 0.0)
        rid = i * BR + lax.broadcasted_iota(jnp.int32, (BR, Din), 0)
        x2 = jnp.where(rid < N, x2, 0.0)
        h = jnp.dot(x2, w_ref[...], preferred_element_type=jnp.float32)
        out_ref[...] = h * c_ref[...] * di

    return pl.pallas_call(
        body,
        grid=(NP // BR,),
        in_specs=[
            pl.BlockSpec((BR, DH), lambda i: (i, 0)),
            pl.BlockSpec((BR, DH), lambda i: (i, 0)),
            pl.BlockSpec((BR, DH), lambda i: (i, 0)),
            pl.BlockSpec((BR, DH), lambda i: (i, 0)),
            pl.BlockSpec((BR, 1), lambda i: (i, 0)),
            pl.BlockSpec((1, Din), lambda i: (0, 0)),
            pl.BlockSpec((Din, Dout), lambda i: (0, 0)),
            pl.BlockSpec((1, Dout), lambda i: (0, 0)),
        ],
        out_specs=pl.BlockSpec((BR, Dout), lambda i: (i, 0)),
        out_shape=jax.ShapeDtypeStruct((NP, Dout), jnp.float32),
    )(za, zb, hsa, hsb, dinv, dvec, W, cvec)


def _tc_final(za, zb, hs2, dinv, dvec, N, BC):
    D = hs2.shape[1]

    def body(za_ref, zb_ref, hs_ref, dinv_ref, d_ref, out_ref):
        out_ref[...] = ((za_ref[...] + zb_ref[...] + hs_ref[...])
                        * dinv_ref[...] + d_ref[...])

    return pl.pallas_call(
        body,
        grid=(N // BC,),
        in_specs=[
            pl.BlockSpec((BC, D), lambda i: (i, 0)),
            pl.BlockSpec((BC, D), lambda i: (i, 0)),
            pl.BlockSpec((BC, D), lambda i: (i, 0)),
            pl.BlockSpec((BC, 1), lambda i: (i, 0)),
            pl.BlockSpec((1, D), lambda i: (0, 0)),
        ],
        out_specs=pl.BlockSpec((BC, D), lambda i: (i, 0)),
        out_shape=jax.ShapeDtypeStruct((N, D), jnp.float32),
    )(za, zb, hs2, dinv, dvec)


def kernel(x, edge_index, W1, b1, g1, beta1, rm1, rv1,
           W2, b2, g2, beta2, rm2, rv2):
    N, Din = x.shape
    D1 = W1.shape[1]
    D2 = W2.shape[1]
    E = edge_index.shape[1]
    NC, NSUB = _sc_geometry()
    NW = NC * NSUB

    unit = NSUB * CH
    NP = -(-(N + 1) // unit) * unit
    NB2 = -(-E // (NW * K))
    NB1 = NC * NB2
    EP = NW * NB2 * K

    src = edge_index[0].astype(jnp.int32)
    dst = edge_index[1].astype(jnp.int32)
    pad = jnp.full((EP - E,), N, jnp.int32)
    src_p = jnp.concatenate([src, pad])
    dst_p = jnp.concatenate([dst, pad])
    x_p = jnp.pad(x, ((0, NP - N), (0, 0)))

    c1 = g1 * lax.rsqrt(rv1 + EPS)
    d1 = (beta1 + (b1 - rm1) * c1)[None, :]
    c1 = c1[None, :]
    c2 = g2 * lax.rsqrt(rv2 + EPS)
    d2 = (beta2 + (b2 - rm2) * c2)[None, :]
    c2 = c2[None, :]

    zeros_np = jnp.zeros((NP,), jnp.float32)
    zrows1 = jnp.zeros((CH, D1 // NC), jnp.float32)
    zrows2 = jnp.zeros((CH, D2), jnp.float32)

    BR = NP // 10 if NP % 10 == 0 else NP // 8
    BC = N // 10 if N % 10 == 0 else N // 8

    hists = _make_degree_kernel(NC, NSUB, NP, NB2 * K)(
        dst_p.reshape(NW, NB2 * K), zeros_np)
    hsa, hsb, dinv = _tc_deg_matmul(x_p, W1, c1, hists, NW, BR)
    z1 = _make_agg_colsplit_kernel(NC, NSUB, NP, D1 // NC, NB1)(
        src_p.reshape(NSUB, NB1, K), dst_p.reshape(NSUB, NB1, K),
        hsa, hsb, zrows1)
    hs2 = _tc_mid(z1[0], z1[1], hsa, hsb, dinv, d1, W2, c2, N, BR)
    z2 = _make_agg_kernel(NC, NSUB, NP, D2, NB2)(
        src_p.reshape(NW, NB2, K), dst_p.reshape(NW, NB2, K), hs2, zrows2)
    return _tc_final(z2[0], z2[1], hs2, dinv, d2, N, BC)
